# baseline (speedup 1.0000x reference)
import functools

import jax
import jax.numpy as jnp
from jax import lax
from jax.experimental import pallas as pl
from jax.experimental.pallas import tpu as pltpu
from jax.experimental.pallas import tpu_sc as plsc

_T_RANGE, _N_MCC, _N_CUR = 366, 1000, 60
_D_T, _D_MCC, _D_CUR = 32, 64, 16
_PROJ = 256
_PROJ_W = _PROJ // 2
_OFF_CUR = 1000
_OFF_TIME = 1064
_P_ROWS = 1432

_NC, _NS = 2, 16
_NW = _NC * _NS
_C = 80

_M_TC = 512
_N_TC = 122880


def _proj_body(time_ref, mcc_ref, cur_ref, w_ref, b_ref, out_ref):
    wm = w_ref[0:_D_MCC, :]
    wc = w_ref[_D_MCC:_D_MCC + _D_CUR, :]
    wt = w_ref[_D_MCC + _D_CUR:, :]
    out_ref[0:_OFF_CUR, :] = jnp.dot(
        mcc_ref[...], wm, preferred_element_type=jnp.float32)
    out_ref[_OFF_CUR:_OFF_TIME, :] = jnp.dot(
        cur_ref[...], wc, preferred_element_type=jnp.float32) + b_ref[...]
    out_ref[_OFF_TIME:_P_ROWS, :] = jnp.dot(
        time_ref[...], wt, preferred_element_type=jnp.float32)


def _project_tables(time_table, mcc_table, cur_table, W, b):
    time_pad = jnp.zeros((_P_ROWS - _OFF_TIME, _D_T), jnp.float32).at[:_T_RANGE].set(time_table)
    cur_pad = jnp.zeros((_OFF_TIME - _OFF_CUR, _D_CUR), jnp.float32).at[:_N_CUR].set(cur_table)
    return pl.pallas_call(
        _proj_body,
        out_shape=jax.ShapeDtypeStruct((_P_ROWS, _PROJ), jnp.float32),
    )(time_pad, mcc_table, cur_pad, W, b.reshape(1, _PROJ))


def _sc_body(n_sc, p_hbm, et_hbm, mc_hbm, cu_hbm, out_hbm,
             ei_v, mi_v, ci_v,
             im0, im1, ict0, ict1, gm0, gm1, g0, g1, o0, o1,
             sgm0, sgm1, sgc0, sgc1, so0, so1):
    per_w = n_sc // _NW
    n_chunks = per_w // _C
    n_pairs = n_chunks // 2
    wid = lax.axis_index("s") * _NC + lax.axis_index("c")
    wbase = wid * per_w

    im = (im0, im1)
    ict = (ict0, ict1)
    gm = (gm0, gm1)
    g = (g0, g1)
    o = (o0, o1)
    sgm = (sgm0, sgm1)
    sgc = (sgc0, sgc1)
    so = (so0, so1)

    pltpu.sync_copy(et_hbm.at[pl.ds(wbase, per_w)], ei_v)
    pltpu.sync_copy(mc_hbm.at[pl.ds(wbase, per_w)], mi_v)
    pltpu.sync_copy(cu_hbm.at[pl.ds(wbase, per_w)], ci_v)

    def build_idx(cc, b):
        base = cc * _C
        for j in range(_C // 16):
            src = pl.ds(base + j * 16, 16)
            dst = pl.ds(j * 16, 16)
            im[b][dst] = jnp.clip(mi_v[src], 0, _N_MCC - 1)
            ict[b][dst] = jnp.clip(ci_v[src], 0, _N_CUR - 1) + _OFF_CUR
            ict[b][pl.ds(_C + j * 16, 16)] = (
                jnp.clip(ei_v[src], 0, _T_RANGE - 1) + _OFF_TIME)

    def start_gathers(b):
        pltpu.make_async_copy(p_hbm.at[im[b]], gm[b], sgm[b]).start()
        pltpu.make_async_copy(p_hbm.at[ict[b]], g[b], sgc[b]).start()

    def wait_gathers(b):
        pltpu.make_async_copy(p_hbm.at[im[b]], gm[b], sgm[b]).wait()
        pltpu.make_async_copy(p_hbm.at[ict[b]], g[b], sgc[b]).wait()

    def accum(b):
        hi_mask = jnp.full((16,), -65536, jnp.int32)

        def expand(v):
            a = plsc.bitcast(lax.shift_left(v, 16), jnp.float32)
            c = plsc.bitcast(lax.bitwise_and(v, hi_mask), jnp.float32)
            return a, c

        def tb(t, carry):
            for k in range(_PROJ_W // 16):
                sl = pl.ds(k * 16, 16)
                ma, mb = expand(gm[b][t, sl])
                ca, cb = expand(g[b][t, sl])
                ta, tb2 = expand(g[b][_C + t, sl])
                o[b][t, pl.ds(k * 32, 16)] = ma + ca + ta
                o[b][t, pl.ds(k * 32 + 16, 16)] = mb + cb + tb2
            return carry
        lax.fori_loop(0, _C, tb, 0)

    def start_out(cc, b):
        pltpu.make_async_copy(
            o[b], out_hbm.at[pl.ds(wbase + cc * _C, _C)], so[b]).start()

    def wait_out(b):
        pltpu.make_async_copy(
            o[b], out_hbm.at[pl.ds(wbase, _C)], so[b]).wait()

    build_idx(0, 0)
    start_gathers(0)

    def pair(i, carry):
        build_idx(2 * i + 1, 1)
        start_gathers(1)
        wait_gathers(0)

        @pl.when(i > 0)
        def _():
            wait_out(0)
        accum(0)
        start_out(2 * i, 0)

        @pl.when(i < n_pairs - 1)
        def _():
            build_idx(2 * i + 2, 0)
            start_gathers(0)
        wait_gathers(1)

        @pl.when(i > 0)
        def _():
            wait_out(1)
        accum(1)
        start_out(2 * i + 1, 1)
        return carry

    lax.fori_loop(0, n_pairs, pair, 0)
    wait_out(0)
    wait_out(1)


def _oh_body(et_ref, mc_ref, cu_ref, pm_ref, pc_ref, pt_ref, out_ref):
    mcb = jnp.clip(mc_ref[0, 0, :], 0, _N_MCC - 1)
    cub = jnp.clip(cu_ref[0, 0, :], 0, _N_CUR - 1)
    etb = jnp.clip(et_ref[0, 0, :], 0, _T_RANGE - 1)
    oh_m = (mcb[:, None] == lax.broadcasted_iota(
        jnp.int32, (_M_TC, _OFF_CUR), 1)).astype(jnp.bfloat16)
    oh_c = (cub[:, None] == lax.broadcasted_iota(
        jnp.int32, (_M_TC, _OFF_TIME - _OFF_CUR), 1)).astype(jnp.bfloat16)
    oh_t = (etb[:, None] == lax.broadcasted_iota(
        jnp.int32, (_M_TC, _P_ROWS - _OFF_TIME), 1)).astype(jnp.bfloat16)
    acc = jnp.dot(oh_m, pm_ref[...], preferred_element_type=jnp.float32)
    acc += jnp.dot(oh_c, pc_ref[...], preferred_element_type=jnp.float32)
    acc += jnp.dot(oh_t, pt_ref[...], preferred_element_type=jnp.float32)
    out_ref[...] = acc


def _onehot_tc(P16, et, mc, cu):
    nb = _N_TC // _M_TC
    idx3 = lambda a: a.reshape(nb, 1, _M_TC)
    iblock = pl.BlockSpec((1, 1, _M_TC), lambda bb: (bb, 0, 0))
    return pl.pallas_call(
        _oh_body,
        grid=(nb,),
        in_specs=[
            iblock, iblock, iblock,
            pl.BlockSpec((_OFF_CUR, _PROJ), lambda bb: (0, 0)),
            pl.BlockSpec((_OFF_TIME - _OFF_CUR, _PROJ), lambda bb: (0, 0)),
            pl.BlockSpec((_P_ROWS - _OFF_TIME, _PROJ), lambda bb: (0, 0)),
        ],
        out_specs=pl.BlockSpec((_M_TC, _PROJ), lambda bb: (bb, 0)),
        out_shape=jax.ShapeDtypeStruct((_N_TC, _PROJ), jnp.float32),
    )(idx3(et), idx3(mc), idx3(cu),
      P16[0:_OFF_CUR], P16[_OFF_CUR:_OFF_TIME], P16[_OFF_TIME:_P_ROWS])


def _gather_sum(Pi, et, mc, cu, n_tokens, n_sc):
    per_w = n_sc // _NW
    mesh = plsc.VectorSubcoreMesh(
        core_axis_name="c", subcore_axis_name="s",
        num_cores=_NC, num_subcores=_NS)
    return pl.kernel(
        functools.partial(_sc_body, n_sc),
        out_type=jax.ShapeDtypeStruct((n_tokens, _PROJ), jnp.float32),
        mesh=mesh,
        compiler_params=pltpu.CompilerParams(needs_layout_passes=False),
        scratch_types=[
            pltpu.VMEM((per_w,), jnp.int32),
            pltpu.VMEM((per_w,), jnp.int32),
            pltpu.VMEM((per_w,), jnp.int32),
            pltpu.VMEM((_C,), jnp.int32),
            pltpu.VMEM((_C,), jnp.int32),
            pltpu.VMEM((2 * _C,), jnp.int32),
            pltpu.VMEM((2 * _C,), jnp.int32),
            pltpu.VMEM((_C, _PROJ_W), jnp.int32),
            pltpu.VMEM((_C, _PROJ_W), jnp.int32),
            pltpu.VMEM((2 * _C, _PROJ_W), jnp.int32),
            pltpu.VMEM((2 * _C, _PROJ_W), jnp.int32),
            pltpu.VMEM((_C, _PROJ), jnp.float32),
            pltpu.VMEM((_C, _PROJ), jnp.float32),
            pltpu.SemaphoreType.DMA,
            pltpu.SemaphoreType.DMA,
            pltpu.SemaphoreType.DMA,
            pltpu.SemaphoreType.DMA,
            pltpu.SemaphoreType.DMA,
            pltpu.SemaphoreType.DMA,
        ],
    )(Pi, et, mc, cu)


def kernel(event_time, mcc_code, currency, seq_lens, time_table, mcc_table,
           cur_table, W, b):
    B, S = event_time.shape
    n_tokens = B * S
    P = _project_tables(time_table, mcc_table, cur_table, W, b)
    Pp = P.reshape(_P_ROWS, 8, 2, 16).transpose(0, 1, 3, 2).reshape(_P_ROWS, _PROJ)
    Pi = lax.bitcast_convert_type(
        Pp.astype(jnp.bfloat16).reshape(_P_ROWS, _PROJ_W, 2), jnp.int32)
    P16 = P.astype(jnp.bfloat16)
    et = event_time.reshape(n_tokens).astype(jnp.int32)
    mc = mcc_code.reshape(n_tokens).astype(jnp.int32)
    cu = currency.reshape(n_tokens).astype(jnp.int32)
    n_sc = n_tokens - _N_TC
    out_sc = _gather_sum(Pi, et, mc, cu, n_tokens, n_sc)
    out_tc = _onehot_tc(P16, et[n_sc:], mc[n_sc:], cu[n_sc:])
    out = lax.dynamic_update_slice(out_sc, out_tc, (n_sc, 0))
    return out.reshape(B, S, _PROJ)


# speedup vs baseline: 1.0777x; 1.0777x over previous
import functools

import jax
import jax.numpy as jnp
from jax import lax
from jax.experimental import pallas as pl
from jax.experimental.pallas import tpu as pltpu
from jax.experimental.pallas import tpu_sc as plsc

_T_RANGE, _N_MCC, _N_CUR = 366, 1000, 60
_D_T, _D_MCC, _D_CUR = 32, 64, 16
_PROJ = 256
_PROJ_W = _PROJ // 2
_OFF_CUR = 1000
_OFF_TIME = 1064
_P_ROWS = 1432

_NC, _NS = 2, 16
_NW = _NC * _NS
_C = 80

_M_TC = 512
_N_TC = 122880


def _proj_body(time_ref, mcc_ref, cur_ref, w_ref, b_ref, out_ref):
    wm = w_ref[0:_D_MCC, :]
    wc = w_ref[_D_MCC:_D_MCC + _D_CUR, :]
    wt = w_ref[_D_MCC + _D_CUR:, :]
    out_ref[0:_OFF_CUR, :] = jnp.dot(
        mcc_ref[...], wm, preferred_element_type=jnp.float32)
    out_ref[_OFF_CUR:_OFF_TIME, :] = jnp.dot(
        cur_ref[...], wc, preferred_element_type=jnp.float32) + b_ref[...]
    out_ref[_OFF_TIME:_P_ROWS, :] = jnp.dot(
        time_ref[...], wt, preferred_element_type=jnp.float32)


def _project_tables(time_table, mcc_table, cur_table, W, b):
    time_pad = jnp.zeros((_P_ROWS - _OFF_TIME, _D_T), jnp.float32).at[:_T_RANGE].set(time_table)
    cur_pad = jnp.zeros((_OFF_TIME - _OFF_CUR, _D_CUR), jnp.float32).at[:_N_CUR].set(cur_table)
    return pl.pallas_call(
        _proj_body,
        out_shape=jax.ShapeDtypeStruct((_P_ROWS, _PROJ), jnp.float32),
    )(time_pad, mcc_table, cur_pad, W, b.reshape(1, _PROJ))


def _sc_body(n_sc, p_hbm, et_hbm, mc_hbm, cu_hbm, out_hbm,
             ei_v, mi_v, ci_v,
             im0, im1, ict0, ict1, gm0, gm1, g0, g1, o0, o1,
             sgm0, sgm1, sgc0, sgc1, so0, so1):
    per_w = n_sc // _NW
    n_chunks = per_w // _C
    n_pairs = n_chunks // 2
    wid = lax.axis_index("s") * _NC + lax.axis_index("c")
    wbase = wid * per_w

    im = (im0, im1)
    ict = (ict0, ict1)
    gm = (gm0, gm1)
    g = (g0, g1)
    o = (o0, o1)
    sgm = (sgm0, sgm1)
    sgc = (sgc0, sgc1)
    so = (so0, so1)

    pltpu.sync_copy(et_hbm.at[pl.ds(wbase, per_w)], ei_v)
    pltpu.sync_copy(mc_hbm.at[pl.ds(wbase, per_w)], mi_v)
    pltpu.sync_copy(cu_hbm.at[pl.ds(wbase, per_w)], ci_v)

    def build_idx(cc, b):
        base = cc * _C
        for j in range(_C // 16):
            src = pl.ds(base + j * 16, 16)
            dst = pl.ds(j * 16, 16)
            im[b][dst] = jnp.clip(mi_v[src], 0, _N_MCC - 1)
            ict[b][dst] = jnp.clip(ci_v[src], 0, _N_CUR - 1) + _OFF_CUR
            ict[b][pl.ds(_C + j * 16, 16)] = (
                jnp.clip(ei_v[src], 0, _T_RANGE - 1) + _OFF_TIME)

    def start_gathers(b):
        pltpu.make_async_copy(p_hbm.at[im[b]], gm[b], sgm[b]).start()
        pltpu.make_async_copy(p_hbm.at[ict[b]], g[b], sgc[b]).start()

    def wait_gathers(b):
        pltpu.make_async_copy(p_hbm.at[im[b]], gm[b], sgm[b]).wait()
        pltpu.make_async_copy(p_hbm.at[ict[b]], g[b], sgc[b]).wait()

    def accum(b):
        hi_mask = jnp.full((16,), -65536, jnp.int32)

        def expand(v):
            a = plsc.bitcast(lax.shift_left(v, 16), jnp.float32)
            c = plsc.bitcast(lax.bitwise_and(v, hi_mask), jnp.float32)
            return a, c

        def tb(t, carry):
            for k in range(_PROJ_W // 16):
                sl = pl.ds(k * 16, 16)
                ma, mb = expand(gm[b][t, sl])
                ca, cb = expand(g[b][t, sl])
                ta, tb2 = expand(g[b][_C + t, sl])
                o[b][t, pl.ds(k * 32, 16)] = ma + ca + ta
                o[b][t, pl.ds(k * 32 + 16, 16)] = mb + cb + tb2
            return carry
        lax.fori_loop(0, _C, tb, 0)

    def start_out(cc, b):
        pltpu.make_async_copy(
            o[b], out_hbm.at[pl.ds(wbase + cc * _C, _C)], so[b]).start()

    def wait_out(b):
        pltpu.make_async_copy(
            o[b], out_hbm.at[pl.ds(wbase, _C)], so[b]).wait()

    build_idx(0, 0)
    start_gathers(0)

    def pair(i, carry):
        build_idx(2 * i + 1, 1)
        start_gathers(1)
        wait_gathers(0)

        @pl.when(i > 0)
        def _():
            wait_out(0)
        accum(0)
        start_out(2 * i, 0)

        @pl.when(i < n_pairs - 1)
        def _():
            build_idx(2 * i + 2, 0)
            start_gathers(0)
        wait_gathers(1)

        @pl.when(i > 0)
        def _():
            wait_out(1)
        accum(1)
        start_out(2 * i + 1, 1)
        return carry

    lax.fori_loop(0, n_pairs, pair, 0)
    wait_out(0)
    wait_out(1)


def _oh_body(et_ref, mc_ref, cu_ref, pm_ref, pc_ref, pt_ref, out_ref):
    mcb = jnp.clip(mc_ref[0, 0, :], 0, _N_MCC - 1)
    cub = jnp.clip(cu_ref[0, 0, :], 0, _N_CUR - 1)
    etb = jnp.clip(et_ref[0, 0, :], 0, _T_RANGE - 1)
    oh_m = (mcb[:, None] == lax.broadcasted_iota(
        jnp.int32, (_M_TC, _OFF_CUR), 1)).astype(jnp.bfloat16)
    oh_c = (cub[:, None] == lax.broadcasted_iota(
        jnp.int32, (_M_TC, _OFF_TIME - _OFF_CUR), 1)).astype(jnp.bfloat16)
    oh_t = (etb[:, None] == lax.broadcasted_iota(
        jnp.int32, (_M_TC, _P_ROWS - _OFF_TIME), 1)).astype(jnp.bfloat16)
    acc = jnp.dot(oh_m, pm_ref[...], preferred_element_type=jnp.float32)
    acc += jnp.dot(oh_c, pc_ref[...], preferred_element_type=jnp.float32)
    acc += jnp.dot(oh_t, pt_ref[...], preferred_element_type=jnp.float32)
    out_ref[...] = acc


def _onehot_tc(P16, et, mc, cu, n_tokens):
    nb = _N_TC // _M_TC
    head_blocks = (n_tokens - _N_TC) // _M_TC
    idx3 = lambda a: a.reshape(nb, 1, _M_TC)
    iblock = pl.BlockSpec((1, 1, _M_TC), lambda bb: (bb, 0, 0))
    return pl.pallas_call(
        _oh_body,
        grid=(nb,),
        in_specs=[
            iblock, iblock, iblock,
            pl.BlockSpec((_OFF_CUR, _PROJ), lambda bb: (0, 0)),
            pl.BlockSpec((_OFF_TIME - _OFF_CUR, _PROJ), lambda bb: (0, 0)),
            pl.BlockSpec((_P_ROWS - _OFF_TIME, _PROJ), lambda bb: (0, 0)),
        ],
        out_specs=pl.BlockSpec((_M_TC, _PROJ),
                               lambda bb: (bb + head_blocks, 0)),
        out_shape=jax.ShapeDtypeStruct((n_tokens, _PROJ), jnp.float32),
    )(idx3(et), idx3(mc), idx3(cu),
      P16[0:_OFF_CUR], P16[_OFF_CUR:_OFF_TIME], P16[_OFF_TIME:_P_ROWS])


def _gather_sum(Pi, et, mc, cu, n_tokens, n_sc):
    per_w = n_sc // _NW
    mesh = plsc.VectorSubcoreMesh(
        core_axis_name="c", subcore_axis_name="s",
        num_cores=_NC, num_subcores=_NS)
    return pl.kernel(
        functools.partial(_sc_body, n_sc),
        out_type=jax.ShapeDtypeStruct((n_sc, _PROJ), jnp.float32),
        mesh=mesh,
        compiler_params=pltpu.CompilerParams(needs_layout_passes=False),
        scratch_types=[
            pltpu.VMEM((per_w,), jnp.int32),
            pltpu.VMEM((per_w,), jnp.int32),
            pltpu.VMEM((per_w,), jnp.int32),
            pltpu.VMEM((_C,), jnp.int32),
            pltpu.VMEM((_C,), jnp.int32),
            pltpu.VMEM((2 * _C,), jnp.int32),
            pltpu.VMEM((2 * _C,), jnp.int32),
            pltpu.VMEM((_C, _PROJ_W), jnp.int32),
            pltpu.VMEM((_C, _PROJ_W), jnp.int32),
            pltpu.VMEM((2 * _C, _PROJ_W), jnp.int32),
            pltpu.VMEM((2 * _C, _PROJ_W), jnp.int32),
            pltpu.VMEM((_C, _PROJ), jnp.float32),
            pltpu.VMEM((_C, _PROJ), jnp.float32),
            pltpu.SemaphoreType.DMA,
            pltpu.SemaphoreType.DMA,
            pltpu.SemaphoreType.DMA,
            pltpu.SemaphoreType.DMA,
            pltpu.SemaphoreType.DMA,
            pltpu.SemaphoreType.DMA,
        ],
    )(Pi, et, mc, cu)


def kernel(event_time, mcc_code, currency, seq_lens, time_table, mcc_table,
           cur_table, W, b):
    B, S = event_time.shape
    n_tokens = B * S
    P = _project_tables(time_table, mcc_table, cur_table, W, b)
    Pp = P.reshape(_P_ROWS, 8, 2, 16).transpose(0, 1, 3, 2).reshape(_P_ROWS, _PROJ)
    Pi = lax.bitcast_convert_type(
        Pp.astype(jnp.bfloat16).reshape(_P_ROWS, _PROJ_W, 2), jnp.int32)
    P16 = P.astype(jnp.bfloat16)
    et = event_time.reshape(n_tokens).astype(jnp.int32)
    mc = mcc_code.reshape(n_tokens).astype(jnp.int32)
    cu = currency.reshape(n_tokens).astype(jnp.int32)
    n_sc = n_tokens - _N_TC
    out_sc = _gather_sum(Pi, et, mc, cu, n_tokens, n_sc)
    out_tc = _onehot_tc(P16, et[n_sc:], mc[n_sc:], cu[n_sc:], n_tokens)
    out = lax.dynamic_update_slice(out_tc, out_sc, (0, 0))
    return out.reshape(B, S, _PROJ)


# speedup vs baseline: 1.1122x; 1.0321x over previous
import functools

import jax
import jax.numpy as jnp
from jax import lax
from jax.experimental import pallas as pl
from jax.experimental.pallas import tpu as pltpu
from jax.experimental.pallas import tpu_sc as plsc

_T_RANGE, _N_MCC, _N_CUR = 366, 1000, 60
_D_T, _D_MCC, _D_CUR = 32, 64, 16
_PROJ = 256
_PROJ_W = _PROJ // 2
_OFF_CUR = 1000
_OFF_TIME = 1064
_P_ROWS = 1432

_NC, _NS = 2, 16
_NW = _NC * _NS
_C = 80

_M_TC = 512
_N_TC = 133120


def _proj_body(time_ref, mcc_ref, cur_ref, w_ref, b_ref, out_ref):
    wm = w_ref[0:_D_MCC, :]
    wc = w_ref[_D_MCC:_D_MCC + _D_CUR, :]
    wt = w_ref[_D_MCC + _D_CUR:, :]
    out_ref[0:_OFF_CUR, :] = jnp.dot(
        mcc_ref[...], wm, preferred_element_type=jnp.float32)
    out_ref[_OFF_CUR:_OFF_TIME, :] = jnp.dot(
        cur_ref[...], wc, preferred_element_type=jnp.float32) + b_ref[...]
    out_ref[_OFF_TIME:_P_ROWS, :] = jnp.dot(
        time_ref[...], wt, preferred_element_type=jnp.float32)


def _project_tables(time_table, mcc_table, cur_table, W, b):
    time_pad = jnp.zeros((_P_ROWS - _OFF_TIME, _D_T), jnp.float32).at[:_T_RANGE].set(time_table)
    cur_pad = jnp.zeros((_OFF_TIME - _OFF_CUR, _D_CUR), jnp.float32).at[:_N_CUR].set(cur_table)
    return pl.pallas_call(
        _proj_body,
        out_shape=jax.ShapeDtypeStruct((_P_ROWS, _PROJ), jnp.float32),
    )(time_pad, mcc_table, cur_pad, W, b.reshape(1, _PROJ))


def _sc_body(n_sc, p_hbm, et_hbm, mc_hbm, cu_hbm, out_hbm,
             ei_v, mi_v, ci_v,
             im0, im1, ict0, ict1, gm0, gm1, g0, g1, o0, o1,
             sgm0, sgm1, sgc0, sgc1, so0, so1):
    per_w = n_sc // _NW
    n_chunks = per_w // _C
    n_pairs = n_chunks // 2
    wid = lax.axis_index("s") * _NC + lax.axis_index("c")
    wbase = wid * per_w

    im = (im0, im1)
    ict = (ict0, ict1)
    gm = (gm0, gm1)
    g = (g0, g1)
    o = (o0, o1)
    sgm = (sgm0, sgm1)
    sgc = (sgc0, sgc1)
    so = (so0, so1)

    pltpu.sync_copy(et_hbm.at[pl.ds(wbase, per_w)], ei_v)
    pltpu.sync_copy(mc_hbm.at[pl.ds(wbase, per_w)], mi_v)
    pltpu.sync_copy(cu_hbm.at[pl.ds(wbase, per_w)], ci_v)

    def build_idx(cc, b):
        base = cc * _C
        for j in range(_C // 16):
            src = pl.ds(base + j * 16, 16)
            dst = pl.ds(j * 16, 16)
            im[b][dst] = jnp.clip(mi_v[src], 0, _N_MCC - 1)
            ict[b][dst] = jnp.clip(ci_v[src], 0, _N_CUR - 1) + _OFF_CUR
            ict[b][pl.ds(_C + j * 16, 16)] = (
                jnp.clip(ei_v[src], 0, _T_RANGE - 1) + _OFF_TIME)

    def start_gathers(b):
        pltpu.make_async_copy(p_hbm.at[im[b]], gm[b], sgm[b]).start()
        pltpu.make_async_copy(p_hbm.at[ict[b]], g[b], sgc[b]).start()

    def wait_gathers(b):
        pltpu.make_async_copy(p_hbm.at[im[b]], gm[b], sgm[b]).wait()
        pltpu.make_async_copy(p_hbm.at[ict[b]], g[b], sgc[b]).wait()

    def accum(b):
        hi_mask = jnp.full((16,), -65536, jnp.int32)

        def expand(v):
            a = plsc.bitcast(lax.shift_left(v, 16), jnp.float32)
            c = plsc.bitcast(lax.bitwise_and(v, hi_mask), jnp.float32)
            return a, c

        def tb(t, carry):
            for k in range(_PROJ_W // 16):
                sl = pl.ds(k * 16, 16)
                ma, mb = expand(gm[b][t, sl])
                ca, cb = expand(g[b][t, sl])
                ta, tb2 = expand(g[b][_C + t, sl])
                o[b][t, pl.ds(k * 32, 16)] = ma + ca + ta
                o[b][t, pl.ds(k * 32 + 16, 16)] = mb + cb + tb2
            return carry
        lax.fori_loop(0, _C, tb, 0)

    def start_out(cc, b):
        pltpu.make_async_copy(
            o[b], out_hbm.at[pl.ds(wbase + cc * _C, _C)], so[b]).start()

    def wait_out(b):
        pltpu.make_async_copy(
            o[b], out_hbm.at[pl.ds(wbase, _C)], so[b]).wait()

    build_idx(0, 0)
    start_gathers(0)

    def pair(i, carry):
        build_idx(2 * i + 1, 1)
        start_gathers(1)
        wait_gathers(0)

        @pl.when(i > 0)
        def _():
            wait_out(0)
        accum(0)
        start_out(2 * i, 0)

        @pl.when(i < n_pairs - 1)
        def _():
            build_idx(2 * i + 2, 0)
            start_gathers(0)
        wait_gathers(1)

        @pl.when(i > 0)
        def _():
            wait_out(1)
        accum(1)
        start_out(2 * i + 1, 1)
        return carry

    lax.fori_loop(0, n_pairs, pair, 0)
    wait_out(0)
    wait_out(1)


def _oh_body(et_ref, mc_ref, cu_ref, pm_ref, pc_ref, pt_ref, out_ref):
    mcb = jnp.clip(mc_ref[0, 0, :], 0, _N_MCC - 1)
    cub = jnp.clip(cu_ref[0, 0, :], 0, _N_CUR - 1)
    etb = jnp.clip(et_ref[0, 0, :], 0, _T_RANGE - 1)
    oh_m = (mcb[:, None] == lax.broadcasted_iota(
        jnp.int32, (_M_TC, _OFF_CUR), 1)).astype(jnp.bfloat16)
    oh_c = (cub[:, None] == lax.broadcasted_iota(
        jnp.int32, (_M_TC, _OFF_TIME - _OFF_CUR), 1)).astype(jnp.bfloat16)
    oh_t = (etb[:, None] == lax.broadcasted_iota(
        jnp.int32, (_M_TC, _P_ROWS - _OFF_TIME), 1)).astype(jnp.bfloat16)
    acc = jnp.dot(oh_m, pm_ref[...], preferred_element_type=jnp.float32)
    acc += jnp.dot(oh_c, pc_ref[...], preferred_element_type=jnp.float32)
    acc += jnp.dot(oh_t, pt_ref[...], preferred_element_type=jnp.float32)
    out_ref[...] = acc


def _onehot_tc(P16, et, mc, cu, n_tokens):
    nb = _N_TC // _M_TC
    head_blocks = (n_tokens - _N_TC) // _M_TC
    idx3 = lambda a: a.reshape(nb, 1, _M_TC)
    iblock = pl.BlockSpec((1, 1, _M_TC), lambda bb: (bb, 0, 0))
    return pl.pallas_call(
        _oh_body,
        grid=(nb,),
        in_specs=[
            iblock, iblock, iblock,
            pl.BlockSpec((_OFF_CUR, _PROJ), lambda bb: (0, 0)),
            pl.BlockSpec((_OFF_TIME - _OFF_CUR, _PROJ), lambda bb: (0, 0)),
            pl.BlockSpec((_P_ROWS - _OFF_TIME, _PROJ), lambda bb: (0, 0)),
        ],
        out_specs=pl.BlockSpec((_M_TC, _PROJ),
                               lambda bb: (bb + head_blocks, 0)),
        out_shape=jax.ShapeDtypeStruct((n_tokens, _PROJ), jnp.float32),
    )(idx3(et), idx3(mc), idx3(cu),
      P16[0:_OFF_CUR], P16[_OFF_CUR:_OFF_TIME], P16[_OFF_TIME:_P_ROWS])


def _gather_sum(Pi, et, mc, cu, n_tokens, n_sc):
    per_w = n_sc // _NW
    mesh = plsc.VectorSubcoreMesh(
        core_axis_name="c", subcore_axis_name="s",
        num_cores=_NC, num_subcores=_NS)
    return pl.kernel(
        functools.partial(_sc_body, n_sc),
        out_type=jax.ShapeDtypeStruct((n_sc, _PROJ), jnp.float32),
        mesh=mesh,
        compiler_params=pltpu.CompilerParams(needs_layout_passes=False),
        scratch_types=[
            pltpu.VMEM((per_w,), jnp.int32),
            pltpu.VMEM((per_w,), jnp.int32),
            pltpu.VMEM((per_w,), jnp.int32),
            pltpu.VMEM((_C,), jnp.int32),
            pltpu.VMEM((_C,), jnp.int32),
            pltpu.VMEM((2 * _C,), jnp.int32),
            pltpu.VMEM((2 * _C,), jnp.int32),
            pltpu.VMEM((_C, _PROJ_W), jnp.int32),
            pltpu.VMEM((_C, _PROJ_W), jnp.int32),
            pltpu.VMEM((2 * _C, _PROJ_W), jnp.int32),
            pltpu.VMEM((2 * _C, _PROJ_W), jnp.int32),
            pltpu.VMEM((_C, _PROJ), jnp.float32),
            pltpu.VMEM((_C, _PROJ), jnp.float32),
            pltpu.SemaphoreType.DMA,
            pltpu.SemaphoreType.DMA,
            pltpu.SemaphoreType.DMA,
            pltpu.SemaphoreType.DMA,
            pltpu.SemaphoreType.DMA,
            pltpu.SemaphoreType.DMA,
        ],
    )(Pi, et, mc, cu)


def kernel(event_time, mcc_code, currency, seq_lens, time_table, mcc_table,
           cur_table, W, b):
    B, S = event_time.shape
    n_tokens = B * S
    P = _project_tables(time_table, mcc_table, cur_table, W, b)
    Pp = P.reshape(_P_ROWS, 8, 2, 16).transpose(0, 1, 3, 2).reshape(_P_ROWS, _PROJ)
    Pi = lax.bitcast_convert_type(
        Pp.astype(jnp.bfloat16).reshape(_P_ROWS, _PROJ_W, 2), jnp.int32)
    P16 = P.astype(jnp.bfloat16)
    et = event_time.reshape(n_tokens).astype(jnp.int32)
    mc = mcc_code.reshape(n_tokens).astype(jnp.int32)
    cu = currency.reshape(n_tokens).astype(jnp.int32)
    n_sc = n_tokens - _N_TC
    out_sc = _gather_sum(Pi, et, mc, cu, n_tokens, n_sc)
    out_tc = _onehot_tc(P16, et[n_sc:], mc[n_sc:], cu[n_sc:], n_tokens)
    out = lax.dynamic_update_slice(out_tc, out_sc, (0, 0))
    return out.reshape(B, S, _PROJ)
